# baseline (speedup 1.0000x reference)
import functools

import jax
import jax.numpy as jnp
from jax import lax
from jax.experimental import pallas as pl
from jax.experimental.pallas import tpu as pltpu
from jax.experimental.pallas import tpu_sc as plsc

B, S, D = 4, 8192, 768
ROWS = B * S
NW = 32
RPW = ROWS // NW
CH = 32
NCH = RPW // CH
NL = 16
DV = D // NL


def _pe_body(x_hbm, tb_hbm, pos_hbm, time_hbm, out_hbm,
             xbuf, tbuf, pbuf, idxv, sem):
    wid = lax.axis_index("s") * 2 + lax.axis_index("c")
    base = wid * RPW
    sbase = base % S

    def chunk(i, _):
        r0 = base + i * CH
        p0 = sbase + i * CH
        pltpu.sync_copy(x_hbm.at[pl.ds(r0, CH)], xbuf)
        pltpu.sync_copy(tb_hbm.at[pl.ds(r0, CH)], idxv)
        pltpu.sync_copy(pos_hbm.at[pl.ds(p0, CH)], pbuf)
        pltpu.async_copy(time_hbm.at[idxv], tbuf, sem).wait()

        def row(c, _):
            for j in range(DV):
                sl = pl.ds(j * NL, NL)
                plsc.addupdate(xbuf.at[c, sl], tbuf[c, sl])
                plsc.addupdate(xbuf.at[c, sl], pbuf[c, sl])
            return _

        lax.fori_loop(0, CH, row, None)
        pltpu.sync_copy(xbuf, out_hbm.at[pl.ds(r0, CH)])
        return _

    lax.fori_loop(0, NCH, chunk, None)


@jax.jit
def _pe(x2d, tb1d, pos_table, time_table):
    mesh = plsc.VectorSubcoreMesh(core_axis_name="c", subcore_axis_name="s")
    return pl.kernel(
        _pe_body,
        mesh=mesh,
        out_type=jax.ShapeDtypeStruct((ROWS, D), jnp.float32),
        scratch_types=[
            pltpu.VMEM((CH, D), jnp.float32),
            pltpu.VMEM((CH, D), jnp.float32),
            pltpu.VMEM((CH, D), jnp.float32),
            pltpu.VMEM((CH,), jnp.int32),
            pltpu.SemaphoreType.DMA,
        ],
    )(x2d, tb1d, pos_table, time_table)


def kernel(x, time_buckets, pos_table, time_table):
    x2d = x.reshape(ROWS, D)
    tb1d = time_buckets.astype(jnp.int32).reshape(ROWS)
    out = _pe(x2d, tb1d, pos_table, time_table)
    return out.reshape(B, S, D)


# speedup vs baseline: 2.2615x; 2.2615x over previous
import functools

import jax
import jax.numpy as jnp
from jax import lax
from jax.experimental import pallas as pl
from jax.experimental.pallas import tpu as pltpu
from jax.experimental.pallas import tpu_sc as plsc

B, S, D = 4, 8192, 768
ROWS = B * S
NW = 32
RPW = ROWS // NW
CH = 16
NCH = RPW // CH
NL = 16
DV = D // NL


def _pe_body(x_hbm, tb_hbm, pos_hbm, time_hbm, out_hbm,
             xb, tbuf, pb, ob, idxall, semL0, semL1, semS0, semS1):
    wid = lax.axis_index("s") * 2 + lax.axis_index("c")
    base = wid * RPW
    sbase = base % S
    semL = (semL0, semL1)
    semS = (semS0, semS1)

    pltpu.sync_copy(tb_hbm.at[pl.ds(base, RPW)], idxall)

    def load_descs(i, b):
        r0 = base + i * CH
        p0 = sbase + i * CH
        return (
            pltpu.make_async_copy(x_hbm.at[pl.ds(r0, CH)], xb.at[b], semL[b]),
            pltpu.make_async_copy(pos_hbm.at[pl.ds(p0, CH)], pb.at[b], semL[b]),
            pltpu.make_async_copy(
                time_hbm.at[idxall.at[pl.ds(i * CH, CH)]], tbuf.at[b], semL[b]),
        )

    def store_desc(i, b):
        r0 = base + i * CH
        return pltpu.make_async_copy(ob.at[b], out_hbm.at[pl.ds(r0, CH)],
                                     semS[b])

    def issue_loads(i, b):
        for d in load_descs(i, b):
            d.start()

    for b in (0, 1):
        issue_loads(b, b)

    def chunk(i, b):
        for d in load_descs(i, b):
            d.wait()

        @pl.when(i >= 2)
        def _():
            store_desc(i - 2, b).wait()

        x_, t_, p_, o_ = xb.at[b], tbuf.at[b], pb.at[b], ob.at[b]

        def row(c, carry):
            for j in range(DV):
                sl = pl.ds(j * NL, NL)
                o_[c, sl] = x_[c, sl] + t_[c, sl] + p_[c, sl]
            return carry

        lax.fori_loop(0, CH, row, None)
        store_desc(i, b).start()

        @pl.when(i + 2 < NCH)
        def _():
            issue_loads(i + 2, b)

    def outer(g, carry):
        chunk(2 * g, 0)
        chunk(2 * g + 1, 1)
        return carry

    lax.fori_loop(0, NCH // 2, outer, None)
    store_desc(NCH - 2, 0).wait()
    store_desc(NCH - 1, 1).wait()


@jax.jit
def _pe(x2d, tb1d, pos_table, time_table):
    mesh = plsc.VectorSubcoreMesh(core_axis_name="c", subcore_axis_name="s")
    return pl.kernel(
        _pe_body,
        mesh=mesh,
        out_type=jax.ShapeDtypeStruct((ROWS, D), jnp.float32),
        scratch_types=[
            pltpu.VMEM((2, CH, D), jnp.float32),
            pltpu.VMEM((2, CH, D), jnp.float32),
            pltpu.VMEM((2, CH, D), jnp.float32),
            pltpu.VMEM((2, CH, D), jnp.float32),
            pltpu.VMEM((RPW,), jnp.int32),
            pltpu.SemaphoreType.DMA,
            pltpu.SemaphoreType.DMA,
            pltpu.SemaphoreType.DMA,
            pltpu.SemaphoreType.DMA,
        ],
    )(x2d, tb1d, pos_table, time_table)


def kernel(x, time_buckets, pos_table, time_table):
    x2d = x.reshape(ROWS, D)
    tb1d = time_buckets.astype(jnp.int32).reshape(ROWS)
    out = _pe(x2d, tb1d, pos_table, time_table)
    return out.reshape(B, S, D)
